# Initial kernel scaffold; baseline (speedup 1.0000x reference)
import jax
import jax.numpy as jnp
from jax.experimental import pallas as pl


def kernel(boxes3D, classification, poses, confidence):
    raise NotImplementedError("write your pallas kernel here")


# speedup vs baseline: 225.5686x; 225.5686x over previous
import functools

import jax
import jax.numpy as jnp
from jax import lax
from jax.experimental import pallas as pl

_C = 8
_M = 5000
_MP = 5120
_RT = 256
_CB = 512
_NT = _MP // _RT
_NB = _MP // _CB
_SCORE_T = 0.8
_IOU_T = 0.5
_MAXDET = 300
_BIGI = 2**30
_INF = float("inf")


def _heads_body(score_r, xmin_r, ymin_r, xmax_r, ymax_r,
                score_c, xmin_c, ymin_c, xmax_c, ymax_c, out):
    t = pl.program_id(1)
    rs = score_r[0, 0, :]
    rxm = xmin_r[0, 0, :]
    rym = ymin_r[0, 0, :]
    rxM = xmax_r[0, 0, :]
    ryM = ymax_r[0, 0, :]
    area_i = (rxM - rxm + 1.0) * (ryM - rym + 1.0)
    nb = (t * _RT + _RT + _CB - 1) // _CB

    def body(b, acc):
        sl = pl.ds(b * _CB, _CB)
        cs = score_c[sl, :]
        cxm = xmin_c[sl, :]
        cym = ymin_c[sl, :]
        cxM = xmax_c[sl, :]
        cyM = ymax_c[sl, :]
        x1 = jnp.maximum(cxm, rxm)
        y1 = jnp.maximum(cym, rym)
        x2 = jnp.minimum(cxM, rxM)
        y2 = jnp.minimum(cyM, ryM)
        wid = x2 - x1 + 1.0
        hei = y2 - y1 + 1.0
        inter = wid * hei
        area_j = (cxM - cxm + 1.0) * (cyM - cym + 1.0)
        den = area_j + area_i - inter
        ov = jnp.where(den == 0.0, 0.0, inter / jnp.where(den == 0.0, 1.0, den))
        ov = jnp.where(wid <= 0.0, 0.0, ov)
        ov = jnp.where(hei <= 0.0, 0.0, ov)
        ind = (ov > _IOU_T) & (cs > _SCORE_T)
        jv = lax.broadcasted_iota(jnp.int32, (_CB, _RT), 0) + b * _CB
        cand = jnp.where(ind, jv, _BIGI)
        return jnp.minimum(acc, jnp.min(cand, axis=0))

    acc = lax.fori_loop(0, nb, body, jnp.full((_RT,), _BIGI, jnp.int32))
    out[0, 0, :] = jnp.where(rs > _SCORE_T, acc, -1)


def _ins3(ak1, aj1, ak2, aj2, ak3, aj3, k, j):
    lt1 = k < ak1
    lt2 = k < ak2
    lt3 = k < ak3
    nk3 = jnp.where(lt3, jnp.where(lt2, ak2, k), ak3)
    nj3 = jnp.where(lt3, jnp.where(lt2, aj2, j), aj3)
    nk2 = jnp.where(lt2, jnp.where(lt1, ak1, k), ak2)
    nj2 = jnp.where(lt2, jnp.where(lt1, aj1, j), aj2)
    nk1 = jnp.where(lt1, k, ak1)
    nj1 = jnp.where(lt1, j, aj1)
    return nk1, nj1, nk2, nj2, nk3, nj3


def _select_body(score_r, head_c, cg_c, cnt,
                 j1o, j2o, j3o, s1o, s2o, s3o, keepo):
    t = pl.program_id(1)
    rs = score_r[0, 0, :]

    def body(b, carry):
        ak1, aj1, ak2, aj2, ak3, aj3 = carry
        sl = pl.ds(b * _CB, _CB)
        hd = head_c[sl, :]
        cg = cg_c[sl, :]
        iv = lax.broadcasted_iota(jnp.int32, (_CB, _RT), 1) + t * _RT
        elig = (hd == iv) & (cg != 0.0)
        key = jnp.where(elig, cg, _INF)
        rv = lax.broadcasted_iota(jnp.int32, (_CB, _RT), 0) + b * _CB
        k1 = jnp.min(key, axis=0)
        j1 = jnp.min(jnp.where(key == k1, rv, _BIGI), axis=0)
        key = jnp.where(rv == j1, _INF, key)
        k2 = jnp.min(key, axis=0)
        j2 = jnp.min(jnp.where(key == k2, rv, _BIGI), axis=0)
        key = jnp.where(rv == j2, _INF, key)
        k3 = jnp.min(key, axis=0)
        j3 = jnp.min(jnp.where(key == k3, rv, _BIGI), axis=0)
        c = _ins3(ak1, aj1, ak2, aj2, ak3, aj3, k1, j1)
        c = _ins3(*c, k2, j2)
        c = _ins3(*c, k3, j3)
        return c

    init = (jnp.full((_RT,), _INF, jnp.float32), jnp.full((_RT,), _BIGI, jnp.int32),
            jnp.full((_RT,), _INF, jnp.float32), jnp.full((_RT,), _BIGI, jnp.int32),
            jnp.full((_RT,), _INF, jnp.float32), jnp.full((_RT,), _BIGI, jnp.int32))
    ak1, aj1, ak2, aj2, ak3, aj3 = lax.fori_loop(t * _RT // _CB, _NB, body, init)

    count = cnt[0, 0]
    repeats = jnp.minimum(3, count)
    s1 = (ak1 < _INF) & (repeats >= 1)
    s2 = (ak2 < _INF) & (repeats >= 2)
    s3 = (ak3 < _INF) & (repeats >= 3)
    den = s1.astype(jnp.int32) + s2.astype(jnp.int32) + s3.astype(jnp.int32)
    keep = (rs > _SCORE_T) & (den > 0)
    j1o[0, 0, :] = jnp.where(s1, aj1, 0)
    j2o[0, 0, :] = jnp.where(s2, aj2, 0)
    j3o[0, 0, :] = jnp.where(s3, aj3, 0)
    s1o[0, 0, :] = s1.astype(jnp.int32)
    s2o[0, 0, :] = s2.astype(jnp.int32)
    s3o[0, 0, :] = s3.astype(jnp.int32)
    keepo[0, 0, :] = keep.astype(jnp.int32)


def _tile3(spec_kind):
    return pl.BlockSpec((1, 1, _RT), lambda c, t: (c * _NT + t, 0, 0))


def _col_spec(shape):
    return pl.BlockSpec(shape, lambda c, t: (c, 0, 0))


def _r3(x):
    return x.reshape(_C * _NT, 1, _RT)


def kernel(boxes3D, classification, poses, confidence):
    M, C = _M, _C
    cls = classification.reshape(M, C)
    b3 = boxes3D.reshape(M, C, 16)
    pos = poses.reshape(M, C, 12)
    cf = confidence.reshape(M, C)

    scores_t = jnp.full((C, _MP), -1.0, jnp.float32).at[:, :M].set(cls.T)
    ev = b3[:, :, 0::2]
    od = b3[:, :, 1::2]
    pad = jnp.zeros((C, _MP - M), jnp.float32)

    def padt(x):
        return jnp.concatenate([x.T, pad], axis=1)

    xmin = padt(jnp.min(ev, axis=2))
    ymin = padt(jnp.min(od, axis=2))
    xmax = padt(jnp.max(ev, axis=2))
    ymax = padt(jnp.max(od, axis=2))
    cg = padt(cf)
    counts = jnp.sum(cls > _SCORE_T, axis=0).astype(jnp.int32)

    col = lambda x: x[:, :, None]
    rows = _r3

    grid = (C, _NT)
    heads = pl.pallas_call(
        _heads_body,
        grid=grid,
        in_specs=[_tile3("r")] * 5 + [_col_spec((None, _MP, 1))] * 5,
        out_specs=_tile3("o"),
        out_shape=jax.ShapeDtypeStruct((C * _NT, 1, _RT), jnp.int32),
    )(rows(scores_t), rows(xmin), rows(ymin), rows(xmax), rows(ymax),
      col(scores_t), col(xmin), col(ymin), col(xmax), col(ymax))

    head_c = heads.reshape(C, _MP)

    outs3 = [jax.ShapeDtypeStruct((C * _NT, 1, _RT), jnp.int32)] * 7
    sel = pl.pallas_call(
        _select_body,
        grid=grid,
        in_specs=[_tile3("r"), _col_spec((None, _MP, 1)), _col_spec((None, _MP, 1)),
                  pl.BlockSpec((None, 1, 1), lambda c, t: (c, 0, 0))],
        out_specs=[_tile3("o")] * 7,
        out_shape=outs3,
    )(rows(scores_t), col(head_c), col(cg), counts.reshape(C, 1, 1))
    j1, j2, j3, s1, s2, s3, keep = [x.reshape(C, _MP)[:, :M] for x in sel]

    pos_t = pos.transpose(1, 0, 2)
    g = lambda jx: jnp.take_along_axis(pos_t, jx[:, :, None], axis=1)
    p1 = g(j1) * s1[:, :, None]
    p2 = g(j2) * s2[:, :, None]
    p3 = g(j3) * s3[:, :, None]
    den = (s1 + s2 + s3).astype(jnp.float32)
    pout = jnp.where(den[:, :, None] == 0.0, 0.0,
                     (p1 + p2 + p3) / jnp.where(den[:, :, None] == 0.0, 1.0, den[:, :, None]))

    kb = keep.astype(jnp.bool_)
    keep_all = jnp.concatenate([kb, (counts == 0)[:, None]], axis=1)
    sc = jnp.where(kb, cls.T[:, :M], -jnp.inf)
    sent_sc = jnp.full((C, 1), cls[M - 1, C - 1], jnp.float32)
    scores_all = jnp.concatenate([jnp.where(keep_all[:, :M], sc, -jnp.inf), sent_sc], axis=1)
    lab = jnp.broadcast_to(jnp.arange(C, dtype=jnp.int32)[:, None], (C, M))
    labels_all = jnp.concatenate([lab, jnp.full((C, 1), -1, jnp.int32)], axis=1)
    idxv = jnp.broadcast_to(jnp.arange(M, dtype=jnp.int32)[None, :], (C, M))
    idx_all = jnp.concatenate([idxv, jnp.full((C, 1), -1, jnp.int32)], axis=1)
    poses_all = jnp.concatenate([pout, jnp.full((C, 1, 12), -1.0, jnp.float32)], axis=1)

    keep_f = keep_all.reshape(-1)
    scores_f = scores_all.reshape(-1)
    labels_f = labels_all.reshape(-1)
    idx_f = idx_all.reshape(-1)
    poses_f = poses_all.reshape(-1, 12)

    kint = keep_f.astype(jnp.int32)
    total = jnp.sum(kint)
    masked = jnp.where(keep_f, scores_f, -jnp.inf)
    top, _ = lax.top_k(masked, _MAXDET)
    scores_out = jnp.where(jnp.arange(_MAXDET) < total, top, -1.0)
    rank = jnp.cumsum(kint) - kint
    slot = jnp.where(keep_f, rank, _MAXDET)
    labels_out = jnp.full((_MAXDET,), -1, jnp.int32).at[slot].set(labels_f, mode='drop')
    idx_out = jnp.full((_MAXDET,), -1, jnp.int32).at[slot].set(idx_f, mode='drop')
    poses_out = jnp.full((_MAXDET, 12), -1.0, jnp.float32).at[slot].set(poses_f, mode='drop')
    return scores_out, labels_out, poses_out, idx_out


# speedup vs baseline: 258.2521x; 1.1449x over previous
import functools

import jax
import jax.numpy as jnp
from jax import lax
from jax.experimental import pallas as pl
from jax.experimental.pallas import tpu as pltpu
from jax.experimental.pallas import tpu_sc as plsc

_C = 8
_M = 5000
_MP = 5120
_RT = 256
_CB = 512
_NT = _MP // _RT
_NB = _MP // _CB
_SCORE_T = 0.8
_IOU_T = 0.5
_MAXDET = 300
_BIGI = 2**30
_INF = float("inf")


def _heads_body(score_r, xmin_r, ymin_r, xmax_r, ymax_r,
                score_c, xmin_c, ymin_c, xmax_c, ymax_c, out):
    t = pl.program_id(1)
    rs = score_r[0, 0, :]
    rxm = xmin_r[0, 0, :]
    rym = ymin_r[0, 0, :]
    rxM = xmax_r[0, 0, :]
    ryM = ymax_r[0, 0, :]
    area_i = (rxM - rxm + 1.0) * (ryM - rym + 1.0)
    nb = (t * _RT + _RT + _CB - 1) // _CB

    def body(b, acc):
        sl = pl.ds(b * _CB, _CB)
        cs = score_c[sl, :]
        cxm = xmin_c[sl, :]
        cym = ymin_c[sl, :]
        cxM = xmax_c[sl, :]
        cyM = ymax_c[sl, :]
        x1 = jnp.maximum(cxm, rxm)
        y1 = jnp.maximum(cym, rym)
        x2 = jnp.minimum(cxM, rxM)
        y2 = jnp.minimum(cyM, ryM)
        wid = x2 - x1 + 1.0
        hei = y2 - y1 + 1.0
        inter = wid * hei
        area_j = (cxM - cxm + 1.0) * (cyM - cym + 1.0)
        den = area_j + area_i - inter
        ov = jnp.where(den == 0.0, 0.0, inter / jnp.where(den == 0.0, 1.0, den))
        ov = jnp.where(wid <= 0.0, 0.0, ov)
        ov = jnp.where(hei <= 0.0, 0.0, ov)
        ind = (ov > _IOU_T) & (cs > _SCORE_T)
        jv = lax.broadcasted_iota(jnp.int32, (_CB, _RT), 0) + b * _CB
        cand = jnp.where(ind, jv, _BIGI)
        return jnp.minimum(acc, jnp.min(cand, axis=0))

    acc = lax.fori_loop(0, nb, body, jnp.full((_RT,), _BIGI, jnp.int32))
    out[0, 0, :] = jnp.where(rs > _SCORE_T, acc, -1)


def _ins3(ak1, aj1, ak2, aj2, ak3, aj3, k, j):
    lt1 = k < ak1
    lt2 = k < ak2
    lt3 = k < ak3
    nk3 = jnp.where(lt3, jnp.where(lt2, ak2, k), ak3)
    nj3 = jnp.where(lt3, jnp.where(lt2, aj2, j), aj3)
    nk2 = jnp.where(lt2, jnp.where(lt1, ak1, k), ak2)
    nj2 = jnp.where(lt2, jnp.where(lt1, aj1, j), aj2)
    nk1 = jnp.where(lt1, k, ak1)
    nj1 = jnp.where(lt1, j, aj1)
    return nk1, nj1, nk2, nj2, nk3, nj3


def _select_body(score_r, head_c, cg_c, cnt,
                 j1o, j2o, j3o, s1o, s2o, s3o, keepo):
    t = pl.program_id(1)
    rs = score_r[0, 0, :]

    def body(b, carry):
        ak1, aj1, ak2, aj2, ak3, aj3 = carry
        sl = pl.ds(b * _CB, _CB)
        hd = head_c[sl, :]
        cg = cg_c[sl, :]
        iv = lax.broadcasted_iota(jnp.int32, (_CB, _RT), 1) + t * _RT
        elig = (hd == iv) & (cg != 0.0)
        key = jnp.where(elig, cg, _INF)
        rv = lax.broadcasted_iota(jnp.int32, (_CB, _RT), 0) + b * _CB
        k1 = jnp.min(key, axis=0)
        j1 = jnp.min(jnp.where(key == k1, rv, _BIGI), axis=0)
        key = jnp.where(rv == j1, _INF, key)
        k2 = jnp.min(key, axis=0)
        j2 = jnp.min(jnp.where(key == k2, rv, _BIGI), axis=0)
        key = jnp.where(rv == j2, _INF, key)
        k3 = jnp.min(key, axis=0)
        j3 = jnp.min(jnp.where(key == k3, rv, _BIGI), axis=0)
        c = _ins3(ak1, aj1, ak2, aj2, ak3, aj3, k1, j1)
        c = _ins3(*c, k2, j2)
        c = _ins3(*c, k3, j3)
        return c

    init = (jnp.full((_RT,), _INF, jnp.float32), jnp.full((_RT,), _BIGI, jnp.int32),
            jnp.full((_RT,), _INF, jnp.float32), jnp.full((_RT,), _BIGI, jnp.int32),
            jnp.full((_RT,), _INF, jnp.float32), jnp.full((_RT,), _BIGI, jnp.int32))
    ak1, aj1, ak2, aj2, ak3, aj3 = lax.fori_loop(t * _RT // _CB, _NB, body, init)

    count = cnt[0, 0]
    repeats = jnp.minimum(3, count)
    s1 = (ak1 < _INF) & (repeats >= 1)
    s2 = (ak2 < _INF) & (repeats >= 2)
    s3 = (ak3 < _INF) & (repeats >= 3)
    den = s1.astype(jnp.int32) + s2.astype(jnp.int32) + s3.astype(jnp.int32)
    keep = (rs > _SCORE_T) & (den > 0)
    j1o[0, 0, :] = jnp.where(s1, aj1, 0)
    j2o[0, 0, :] = jnp.where(s2, aj2, 0)
    j3o[0, 0, :] = jnp.where(s3, aj3, 0)
    s1o[0, 0, :] = s1.astype(jnp.int32)
    s2o[0, 0, :] = s2.astype(jnp.int32)
    s3o[0, 0, :] = s3.astype(jnp.int32)
    keepo[0, 0, :] = keep.astype(jnp.int32)


_CH = _MP // 4
_TAB = _M * 12


def _sc_posemean_body(pos_hbm, j1h, j2h, j3h, s1h, s2h, s3h, out_hbm,
                      tab_v, j1v, j2v, j3v, s1v, s2v, s3v, out_v):
    wid = lax.axis_index("s") * 2 + lax.axis_index("c")
    c = wid // 4
    q = wid % 4
    base = c * _MP + q * _CH
    pltpu.sync_copy(pos_hbm.at[pl.ds(c * _TAB, _TAB)], tab_v)
    pltpu.sync_copy(j1h.at[pl.ds(base, _CH)], j1v)
    pltpu.sync_copy(j2h.at[pl.ds(base, _CH)], j2v)
    pltpu.sync_copy(j3h.at[pl.ds(base, _CH)], j3v)
    pltpu.sync_copy(s1h.at[pl.ds(base, _CH)], s1v)
    pltpu.sync_copy(s2h.at[pl.ds(base, _CH)], s2v)
    pltpu.sync_copy(s3h.at[pl.ds(base, _CH)], s3v)

    lane = lax.iota(jnp.int32, 16)

    def body(g, carry):
        sl = pl.ds(g * 16, 16)
        jv1 = j1v[sl] * 12
        jv2 = j2v[sl] * 12
        jv3 = j3v[sl] * 12
        sv1 = s1v[sl]
        sv2 = s2v[sl]
        sv3 = s3v[sl]
        den = sv1 + sv2 + sv3
        rden = jnp.where(den == 0.0, 1.0, den)
        zero = den == 0.0
        lv = (g * 16 + lane) * 12
        for d in range(12):
            p1 = plsc.load_gather(tab_v, [jv1 + d])
            p2 = plsc.load_gather(tab_v, [jv2 + d])
            p3 = plsc.load_gather(tab_v, [jv3 + d])
            p = (p1 * sv1 + p2 * sv2 + p3 * sv3) / rden
            p = jnp.where(zero, 0.0, p)
            plsc.store_scatter(out_v, [lv + d], p)
        return carry

    lax.fori_loop(0, _CH // 16, body, 0)
    pltpu.sync_copy(out_v, out_hbm.at[pl.ds(base * 12, _CH * 12)])


@functools.partial(
    pl.kernel,
    out_type=jax.ShapeDtypeStruct((_C * _MP * 12,), jnp.float32),
    mesh=plsc.VectorSubcoreMesh(core_axis_name="c", subcore_axis_name="s"),
    compiler_params=pltpu.CompilerParams(needs_layout_passes=False),
    scratch_types=[
        pltpu.VMEM((_TAB,), jnp.float32),
        pltpu.VMEM((_CH,), jnp.int32),
        pltpu.VMEM((_CH,), jnp.int32),
        pltpu.VMEM((_CH,), jnp.int32),
        pltpu.VMEM((_CH,), jnp.float32),
        pltpu.VMEM((_CH,), jnp.float32),
        pltpu.VMEM((_CH,), jnp.float32),
        pltpu.VMEM((_CH * 12,), jnp.float32),
    ],
)
def _sc_posemean(pos_hbm, j1h, j2h, j3h, s1h, s2h, s3h, out_hbm,
                 tab_v, j1v, j2v, j3v, s1v, s2v, s3v, out_v):
    _sc_posemean_body(pos_hbm, j1h, j2h, j3h, s1h, s2h, s3h, out_hbm,
                      tab_v, j1v, j2v, j3v, s1v, s2v, s3v, out_v)


def _tile3(spec_kind):
    return pl.BlockSpec((1, 1, _RT), lambda c, t: (c * _NT + t, 0, 0))


def _col_spec(shape):
    return pl.BlockSpec(shape, lambda c, t: (c, 0, 0))


def _r3(x):
    return x.reshape(_C * _NT, 1, _RT)


def kernel(boxes3D, classification, poses, confidence):
    M, C = _M, _C
    cls = classification.reshape(M, C)
    b3 = boxes3D.reshape(M, C, 16)
    pos = poses.reshape(M, C, 12)
    cf = confidence.reshape(M, C)

    scores_t = jnp.full((C, _MP), -1.0, jnp.float32).at[:, :M].set(cls.T)
    ev = b3[:, :, 0::2]
    od = b3[:, :, 1::2]
    pad = jnp.zeros((C, _MP - M), jnp.float32)

    def padt(x):
        return jnp.concatenate([x.T, pad], axis=1)

    xmin = padt(jnp.min(ev, axis=2))
    ymin = padt(jnp.min(od, axis=2))
    xmax = padt(jnp.max(ev, axis=2))
    ymax = padt(jnp.max(od, axis=2))
    cg = padt(cf)
    counts = jnp.sum(cls > _SCORE_T, axis=0).astype(jnp.int32)

    col = lambda x: x[:, :, None]
    rows = _r3

    grid = (C, _NT)
    heads = pl.pallas_call(
        _heads_body,
        grid=grid,
        in_specs=[_tile3("r")] * 5 + [_col_spec((None, _MP, 1))] * 5,
        out_specs=_tile3("o"),
        out_shape=jax.ShapeDtypeStruct((C * _NT, 1, _RT), jnp.int32),
    )(rows(scores_t), rows(xmin), rows(ymin), rows(xmax), rows(ymax),
      col(scores_t), col(xmin), col(ymin), col(xmax), col(ymax))

    head_c = heads.reshape(C, _MP)

    outs3 = [jax.ShapeDtypeStruct((C * _NT, 1, _RT), jnp.int32)] * 7
    sel = pl.pallas_call(
        _select_body,
        grid=grid,
        in_specs=[_tile3("r"), _col_spec((None, _MP, 1)), _col_spec((None, _MP, 1)),
                  pl.BlockSpec((None, 1, 1), lambda c, t: (c, 0, 0))],
        out_specs=[_tile3("o")] * 7,
        out_shape=outs3,
    )(rows(scores_t), col(head_c), col(cg), counts.reshape(C, 1, 1))
    j1, j2, j3, s1, s2, s3, keep = [x.reshape(C, _MP) for x in sel]
    keep = keep[:, :M]

    pos_flat = pos.transpose(1, 0, 2).reshape(-1)
    sf = lambda x: x.astype(jnp.float32).reshape(-1)
    pout = _sc_posemean(pos_flat,
                        j1.reshape(-1), j2.reshape(-1), j3.reshape(-1),
                        sf(s1), sf(s2), sf(s3))
    pout = pout.reshape(C, _MP, 12)[:, :M]

    kb = keep.astype(jnp.bool_)
    keep_all = jnp.concatenate([kb, (counts == 0)[:, None]], axis=1)
    sc = jnp.where(kb, cls.T[:, :M], -jnp.inf)
    sent_sc = jnp.full((C, 1), cls[M - 1, C - 1], jnp.float32)
    scores_all = jnp.concatenate([jnp.where(keep_all[:, :M], sc, -jnp.inf), sent_sc], axis=1)
    lab = jnp.broadcast_to(jnp.arange(C, dtype=jnp.int32)[:, None], (C, M))
    labels_all = jnp.concatenate([lab, jnp.full((C, 1), -1, jnp.int32)], axis=1)
    idxv = jnp.broadcast_to(jnp.arange(M, dtype=jnp.int32)[None, :], (C, M))
    idx_all = jnp.concatenate([idxv, jnp.full((C, 1), -1, jnp.int32)], axis=1)
    poses_all = jnp.concatenate([pout, jnp.full((C, 1, 12), -1.0, jnp.float32)], axis=1)

    keep_f = keep_all.reshape(-1)
    scores_f = scores_all.reshape(-1)
    labels_f = labels_all.reshape(-1)
    idx_f = idx_all.reshape(-1)
    poses_f = poses_all.reshape(-1, 12)

    kint = keep_f.astype(jnp.int32)
    total = jnp.sum(kint)
    masked = jnp.where(keep_f, scores_f, -jnp.inf)
    top, _ = lax.top_k(masked, _MAXDET)
    scores_out = jnp.where(jnp.arange(_MAXDET) < total, top, -1.0)
    rank = jnp.cumsum(kint) - kint
    slot = jnp.where(keep_f, rank, _MAXDET)
    labels_out = jnp.full((_MAXDET,), -1, jnp.int32).at[slot].set(labels_f, mode='drop')
    idx_out = jnp.full((_MAXDET,), -1, jnp.int32).at[slot].set(idx_f, mode='drop')
    poses_out = jnp.full((_MAXDET, 12), -1.0, jnp.float32).at[slot].set(poses_f, mode='drop')
    return scores_out, labels_out, poses_out, idx_out


# speedup vs baseline: 337.5902x; 1.3072x over previous
import functools

import jax
import jax.numpy as jnp
from jax import lax
from jax.experimental import pallas as pl
from jax.experimental.pallas import tpu as pltpu
from jax.experimental.pallas import tpu_sc as plsc

_C = 8
_M = 5000
_MP = 5120
_RT = 256
_CB = 512
_NT = _MP // _RT
_NB = _MP // _CB
_SCORE_T = 0.8
_IOU_T = 0.5
_MAXDET = 300
_BIGI = 2**30
_INF = float("inf")


def _heads_body(score_r, xmin_r, ymin_r, xmax_r, ymax_r,
                score_c, xmin_c, ymin_c, xmax_c, ymax_c, out):
    t = pl.program_id(1)
    rs = score_r[0, 0, :]
    rxm = xmin_r[0, 0, :]
    rym = ymin_r[0, 0, :]
    rxM = xmax_r[0, 0, :]
    ryM = ymax_r[0, 0, :]
    area_i = (rxM - rxm + 1.0) * (ryM - rym + 1.0)
    nb = (t * _RT + _RT + _CB - 1) // _CB

    def body(b, acc):
        sl = pl.ds(b * _CB, _CB)
        cs = score_c[sl, :]
        cxm = xmin_c[sl, :]
        cym = ymin_c[sl, :]
        cxM = xmax_c[sl, :]
        cyM = ymax_c[sl, :]
        x1 = jnp.maximum(cxm, rxm)
        y1 = jnp.maximum(cym, rym)
        x2 = jnp.minimum(cxM, rxM)
        y2 = jnp.minimum(cyM, ryM)
        wid = x2 - x1 + 1.0
        hei = y2 - y1 + 1.0
        inter = wid * hei
        area_j = (cxM - cxm + 1.0) * (cyM - cym + 1.0)
        den = area_j + area_i - inter
        ov = jnp.where(den == 0.0, 0.0, inter / jnp.where(den == 0.0, 1.0, den))
        ov = jnp.where(wid <= 0.0, 0.0, ov)
        ov = jnp.where(hei <= 0.0, 0.0, ov)
        ind = (ov > _IOU_T) & (cs > _SCORE_T)
        jv = lax.broadcasted_iota(jnp.int32, (_CB, _RT), 0) + b * _CB
        cand = jnp.where(ind, jv, _BIGI)
        return jnp.minimum(acc, jnp.min(cand, axis=0))

    acc = lax.fori_loop(0, nb, body, jnp.full((_RT,), _BIGI, jnp.int32))
    out[0, 0, :] = jnp.where(rs > _SCORE_T, acc, -1)


def _ins3(ak1, aj1, ak2, aj2, ak3, aj3, k, j):
    lt1 = k < ak1
    lt2 = k < ak2
    lt3 = k < ak3
    nk3 = jnp.where(lt3, jnp.where(lt2, ak2, k), ak3)
    nj3 = jnp.where(lt3, jnp.where(lt2, aj2, j), aj3)
    nk2 = jnp.where(lt2, jnp.where(lt1, ak1, k), ak2)
    nj2 = jnp.where(lt2, jnp.where(lt1, aj1, j), aj2)
    nk1 = jnp.where(lt1, k, ak1)
    nj1 = jnp.where(lt1, j, aj1)
    return nk1, nj1, nk2, nj2, nk3, nj3


def _select_body(score_r, head_c, cg_c, cnt, nnz_s, blist_s,
                 j1o, j2o, j3o, s1o, s2o, s3o, keepo):
    c = pl.program_id(0)
    t = pl.program_id(1)
    rs = score_r[0, 0, :]

    def body(k, carry):
        b = blist_s[c, t, k]
        ak1, aj1, ak2, aj2, ak3, aj3 = carry
        sl = pl.ds(b * _CB, _CB)
        hd = head_c[sl, :]
        cg = cg_c[sl, :]
        iv = lax.broadcasted_iota(jnp.int32, (_CB, _RT), 1) + t * _RT
        elig = (hd == iv) & (cg != 0.0)
        key = jnp.where(elig, cg, _INF)
        rv = lax.broadcasted_iota(jnp.int32, (_CB, _RT), 0) + b * _CB
        k1 = jnp.min(key, axis=0)
        j1 = jnp.min(jnp.where(key == k1, rv, _BIGI), axis=0)
        key = jnp.where(rv == j1, _INF, key)
        k2 = jnp.min(key, axis=0)
        j2 = jnp.min(jnp.where(key == k2, rv, _BIGI), axis=0)
        key = jnp.where(rv == j2, _INF, key)
        k3 = jnp.min(key, axis=0)
        j3 = jnp.min(jnp.where(key == k3, rv, _BIGI), axis=0)
        acc3 = _ins3(ak1, aj1, ak2, aj2, ak3, aj3, k1, j1)
        acc3 = _ins3(*acc3, k2, j2)
        acc3 = _ins3(*acc3, k3, j3)
        return acc3

    init = (jnp.full((_RT,), _INF, jnp.float32), jnp.full((_RT,), _BIGI, jnp.int32),
            jnp.full((_RT,), _INF, jnp.float32), jnp.full((_RT,), _BIGI, jnp.int32),
            jnp.full((_RT,), _INF, jnp.float32), jnp.full((_RT,), _BIGI, jnp.int32))
    ak1, aj1, ak2, aj2, ak3, aj3 = lax.fori_loop(0, nnz_s[c, t], body, init)

    count = cnt[0, 0]
    repeats = jnp.minimum(3, count)
    s1 = (ak1 < _INF) & (repeats >= 1)
    s2 = (ak2 < _INF) & (repeats >= 2)
    s3 = (ak3 < _INF) & (repeats >= 3)
    den = s1.astype(jnp.int32) + s2.astype(jnp.int32) + s3.astype(jnp.int32)
    keep = (rs > _SCORE_T) & (den > 0)
    j1o[0, 0, :] = jnp.where(s1, aj1, 0)
    j2o[0, 0, :] = jnp.where(s2, aj2, 0)
    j3o[0, 0, :] = jnp.where(s3, aj3, 0)
    s1o[0, 0, :] = s1.astype(jnp.int32)
    s2o[0, 0, :] = s2.astype(jnp.int32)
    s3o[0, 0, :] = s3.astype(jnp.int32)
    keepo[0, 0, :] = keep.astype(jnp.int32)


_CH = _MP // 4
_TAB = _M * 12


def _sc_posemean_body(pos_hbm, j1h, j2h, j3h, s1h, s2h, s3h, out_hbm,
                      tab_v, j1v, j2v, j3v, s1v, s2v, s3v, out_v):
    wid = lax.axis_index("s") * 2 + lax.axis_index("c")
    c = wid // 4
    q = wid % 4
    base = c * _MP + q * _CH
    pltpu.sync_copy(pos_hbm.at[pl.ds(c * _TAB, _TAB)], tab_v)
    pltpu.sync_copy(j1h.at[pl.ds(base, _CH)], j1v)
    pltpu.sync_copy(j2h.at[pl.ds(base, _CH)], j2v)
    pltpu.sync_copy(j3h.at[pl.ds(base, _CH)], j3v)
    pltpu.sync_copy(s1h.at[pl.ds(base, _CH)], s1v)
    pltpu.sync_copy(s2h.at[pl.ds(base, _CH)], s2v)
    pltpu.sync_copy(s3h.at[pl.ds(base, _CH)], s3v)

    lane = lax.iota(jnp.int32, 16)

    def body(g, carry):
        sl = pl.ds(g * 16, 16)
        jv1 = j1v[sl] * 12
        jv2 = j2v[sl] * 12
        jv3 = j3v[sl] * 12
        sv1 = s1v[sl]
        sv2 = s2v[sl]
        sv3 = s3v[sl]
        den = sv1 + sv2 + sv3
        rden = jnp.where(den == 0.0, 1.0, den)
        zero = den == 0.0
        lv = (g * 16 + lane) * 12
        for d in range(12):
            p1 = plsc.load_gather(tab_v, [jv1 + d])
            p2 = plsc.load_gather(tab_v, [jv2 + d])
            p3 = plsc.load_gather(tab_v, [jv3 + d])
            p = (p1 * sv1 + p2 * sv2 + p3 * sv3) / rden
            p = jnp.where(zero, 0.0, p)
            plsc.store_scatter(out_v, [lv + d], p)
        return carry

    lax.fori_loop(0, _CH // 16, body, 0)
    pltpu.sync_copy(out_v, out_hbm.at[pl.ds(base * 12, _CH * 12)])


@functools.partial(
    pl.kernel,
    out_type=jax.ShapeDtypeStruct((_C * _MP * 12,), jnp.float32),
    mesh=plsc.VectorSubcoreMesh(core_axis_name="c", subcore_axis_name="s"),
    compiler_params=pltpu.CompilerParams(needs_layout_passes=False),
    scratch_types=[
        pltpu.VMEM((_TAB,), jnp.float32),
        pltpu.VMEM((_CH,), jnp.int32),
        pltpu.VMEM((_CH,), jnp.int32),
        pltpu.VMEM((_CH,), jnp.int32),
        pltpu.VMEM((_CH,), jnp.float32),
        pltpu.VMEM((_CH,), jnp.float32),
        pltpu.VMEM((_CH,), jnp.float32),
        pltpu.VMEM((_CH * 12,), jnp.float32),
    ],
)
def _sc_posemean(pos_hbm, j1h, j2h, j3h, s1h, s2h, s3h, out_hbm,
                 tab_v, j1v, j2v, j3v, s1v, s2v, s3v, out_v):
    _sc_posemean_body(pos_hbm, j1h, j2h, j3h, s1h, s2h, s3h, out_hbm,
                      tab_v, j1v, j2v, j3v, s1v, s2v, s3v, out_v)


def _tile3(spec_kind):
    return pl.BlockSpec((1, 1, _RT), lambda c, t: (c * _NT + t, 0, 0))


def _col_spec(shape):
    return pl.BlockSpec(shape, lambda c, t: (c, 0, 0))


def _r3(x):
    return x.reshape(_C * _NT, 1, _RT)


def kernel(boxes3D, classification, poses, confidence):
    M, C = _M, _C
    cls = classification.reshape(M, C)
    b3 = boxes3D.reshape(M, C, 16)
    pos = poses.reshape(M, C, 12)
    cf = confidence.reshape(M, C)

    scores_t = jnp.full((C, _MP), -1.0, jnp.float32).at[:, :M].set(cls.T)
    ev = b3[:, :, 0::2]
    od = b3[:, :, 1::2]
    pad = jnp.zeros((C, _MP - M), jnp.float32)

    def padt(x):
        return jnp.concatenate([x.T, pad], axis=1)

    xmin = padt(jnp.min(ev, axis=2))
    ymin = padt(jnp.min(od, axis=2))
    xmax = padt(jnp.max(ev, axis=2))
    ymax = padt(jnp.max(od, axis=2))
    cg = padt(cf)
    counts = jnp.sum(cls > _SCORE_T, axis=0).astype(jnp.int32)

    col = lambda x: x[:, :, None]
    rows = _r3

    grid = (C, _NT)
    heads = pl.pallas_call(
        _heads_body,
        grid=grid,
        in_specs=[_tile3("r")] * 5 + [_col_spec((None, _MP, 1))] * 5,
        out_specs=_tile3("o"),
        out_shape=jax.ShapeDtypeStruct((C * _NT, 1, _RT), jnp.int32),
    )(rows(scores_t), rows(xmin), rows(ymin), rows(xmax), rows(ymax),
      col(scores_t), col(xmin), col(ymin), col(xmax), col(ymax))

    head_c = heads.reshape(C, _MP)

    t_of = head_c >> 8
    t_blocks = t_of.reshape(C, _NB, _CB)
    hist = jnp.sum(t_blocks[:, :, :, None] == jnp.arange(_NT)[None, None, None, :],
                   axis=2)
    mask_b = (hist > 0).transpose(0, 2, 1)
    nnz = jnp.sum(mask_b, axis=2).astype(jnp.int32)
    blist = jnp.sort(jnp.where(mask_b, jnp.arange(_NB)[None, None, :], _NB),
                     axis=2).astype(jnp.int32)

    outs3 = [jax.ShapeDtypeStruct((C * _NT, 1, _RT), jnp.int32)] * 7
    sel = pl.pallas_call(
        _select_body,
        grid=grid,
        in_specs=[_tile3("r"), _col_spec((None, _MP, 1)), _col_spec((None, _MP, 1)),
                  pl.BlockSpec((None, 1, 1), lambda c, t: (c, 0, 0)),
                  pl.BlockSpec(memory_space=pltpu.SMEM),
                  pl.BlockSpec(memory_space=pltpu.SMEM)],
        out_specs=[_tile3("o")] * 7,
        out_shape=outs3,
    )(rows(scores_t), col(head_c), col(cg), counts.reshape(C, 1, 1), nnz, blist)
    j1, j2, j3, s1, s2, s3, keep = [x.reshape(C, _MP) for x in sel]
    keep = keep[:, :M]

    pos_flat = pos.transpose(1, 0, 2).reshape(-1)
    sf = lambda x: x.astype(jnp.float32).reshape(-1)
    pout = _sc_posemean(pos_flat,
                        j1.reshape(-1), j2.reshape(-1), j3.reshape(-1),
                        sf(s1), sf(s2), sf(s3))
    pout = pout.reshape(C, _MP, 12)[:, :M]

    kb = keep.astype(jnp.bool_)
    keep_all = jnp.concatenate([kb, (counts == 0)[:, None]], axis=1)
    sc = jnp.where(kb, cls.T[:, :M], -jnp.inf)
    sent_sc = jnp.full((C, 1), cls[M - 1, C - 1], jnp.float32)
    scores_all = jnp.concatenate([jnp.where(keep_all[:, :M], sc, -jnp.inf), sent_sc], axis=1)
    lab = jnp.broadcast_to(jnp.arange(C, dtype=jnp.int32)[:, None], (C, M))
    labels_all = jnp.concatenate([lab, jnp.full((C, 1), -1, jnp.int32)], axis=1)
    idxv = jnp.broadcast_to(jnp.arange(M, dtype=jnp.int32)[None, :], (C, M))
    idx_all = jnp.concatenate([idxv, jnp.full((C, 1), -1, jnp.int32)], axis=1)
    poses_all = jnp.concatenate([pout, jnp.full((C, 1, 12), -1.0, jnp.float32)], axis=1)

    keep_f = keep_all.reshape(-1)
    scores_f = scores_all.reshape(-1)
    labels_f = labels_all.reshape(-1)
    idx_f = idx_all.reshape(-1)
    poses_f = poses_all.reshape(-1, 12)

    kint = keep_f.astype(jnp.int32)
    total = jnp.sum(kint)
    masked = jnp.where(keep_f, scores_f, -jnp.inf)
    top, _ = lax.top_k(masked, _MAXDET)
    scores_out = jnp.where(jnp.arange(_MAXDET) < total, top, -1.0)
    rank = jnp.cumsum(kint) - kint
    slot = jnp.where(keep_f, rank, _MAXDET)
    labels_out = jnp.full((_MAXDET,), -1, jnp.int32).at[slot].set(labels_f, mode='drop')
    idx_out = jnp.full((_MAXDET,), -1, jnp.int32).at[slot].set(idx_f, mode='drop')
    poses_out = jnp.full((_MAXDET, 12), -1.0, jnp.float32).at[slot].set(poses_f, mode='drop')
    return scores_out, labels_out, poses_out, idx_out


# speedup vs baseline: 469.5138x; 1.3908x over previous
import functools

import jax
import jax.numpy as jnp
from jax import lax
from jax.experimental import pallas as pl
from jax.experimental.pallas import tpu as pltpu
from jax.experimental.pallas import tpu_sc as plsc

_C = 8
_M = 5000
_MP = 5120
_RT = 256
_CB = 512
_NT = _MP // _RT
_NB = _MP // _CB
_SCORE_T = 0.8
_IOU_T = 0.5
_MAXDET = 300
_BIGI = 2**30
_INF = float("inf")


def _heads_body(score_r, xmin_r, ymin_r, xmax_r, ymax_r,
                score_c, xmin_c, ymin_c, xmax_c, ymax_c, out):
    t = pl.program_id(1)
    rs = score_r[0, 0, :]
    rxm = xmin_r[0, 0, :]
    rym = ymin_r[0, 0, :]
    rxM = xmax_r[0, 0, :]
    ryM = ymax_r[0, 0, :]
    area_i = (rxM - rxm + 1.0) * (ryM - rym + 1.0)
    nb = (t * _RT + _RT + _CB - 1) // _CB

    def cond(carry):
        b, acc = carry
        return (b < nb) & (jnp.max(acc) == _BIGI)

    def body(carry):
        b, acc = carry
        sl = pl.ds(b * _CB, _CB)
        cs = score_c[sl, :]
        cxm = xmin_c[sl, :]
        cym = ymin_c[sl, :]
        cxM = xmax_c[sl, :]
        cyM = ymax_c[sl, :]
        x1 = jnp.maximum(cxm, rxm)
        y1 = jnp.maximum(cym, rym)
        x2 = jnp.minimum(cxM, rxM)
        y2 = jnp.minimum(cyM, ryM)
        wid = x2 - x1 + 1.0
        hei = y2 - y1 + 1.0
        inter = wid * hei
        area_j = (cxM - cxm + 1.0) * (cyM - cym + 1.0)
        den = area_j + area_i - inter
        ov = jnp.where(den == 0.0, 0.0, inter / jnp.where(den == 0.0, 1.0, den))
        ov = jnp.where(wid <= 0.0, 0.0, ov)
        ov = jnp.where(hei <= 0.0, 0.0, ov)
        ind = (ov > _IOU_T) & (cs > _SCORE_T)
        jv = lax.broadcasted_iota(jnp.int32, (_CB, _RT), 0) + b * _CB
        cand = jnp.where(ind, jv, _BIGI)
        return b + 1, jnp.minimum(acc, jnp.min(cand, axis=0))

    acc0 = jnp.where(rs > _SCORE_T, _BIGI, -2).astype(jnp.int32)
    _, acc = lax.while_loop(cond, body, (jnp.int32(0), acc0))
    out[0, 0, :] = jnp.where(rs > _SCORE_T, acc, -1)


def _ins3(ak1, aj1, ak2, aj2, ak3, aj3, k, j):
    lt1 = k < ak1
    lt2 = k < ak2
    lt3 = k < ak3
    nk3 = jnp.where(lt3, jnp.where(lt2, ak2, k), ak3)
    nj3 = jnp.where(lt3, jnp.where(lt2, aj2, j), aj3)
    nk2 = jnp.where(lt2, jnp.where(lt1, ak1, k), ak2)
    nj2 = jnp.where(lt2, jnp.where(lt1, aj1, j), aj2)
    nk1 = jnp.where(lt1, k, ak1)
    nj1 = jnp.where(lt1, j, aj1)
    return nk1, nj1, nk2, nj2, nk3, nj3


def _select_body(score_r, head_c, cg_c, cnt, nnz_s, blist_s,
                 j1o, j2o, j3o, s1o, s2o, s3o, keepo):
    c = pl.program_id(0)
    t = pl.program_id(1)
    rs = score_r[0, 0, :]

    def body(k, carry):
        b = blist_s[c, t, k]
        ak1, aj1, ak2, aj2, ak3, aj3 = carry
        sl = pl.ds(b * _CB, _CB)
        hd = head_c[sl, :]
        cg = cg_c[sl, :]
        iv = lax.broadcasted_iota(jnp.int32, (_CB, _RT), 1) + t * _RT
        elig = (hd == iv) & (cg != 0.0)
        key = jnp.where(elig, cg, _INF)
        rv = lax.broadcasted_iota(jnp.int32, (_CB, _RT), 0) + b * _CB
        k1 = jnp.min(key, axis=0)
        j1 = jnp.min(jnp.where(key == k1, rv, _BIGI), axis=0)
        key = jnp.where(rv == j1, _INF, key)
        k2 = jnp.min(key, axis=0)
        j2 = jnp.min(jnp.where(key == k2, rv, _BIGI), axis=0)
        key = jnp.where(rv == j2, _INF, key)
        k3 = jnp.min(key, axis=0)
        j3 = jnp.min(jnp.where(key == k3, rv, _BIGI), axis=0)
        acc3 = _ins3(ak1, aj1, ak2, aj2, ak3, aj3, k1, j1)
        acc3 = _ins3(*acc3, k2, j2)
        acc3 = _ins3(*acc3, k3, j3)
        return acc3

    init = (jnp.full((_RT,), _INF, jnp.float32), jnp.full((_RT,), _BIGI, jnp.int32),
            jnp.full((_RT,), _INF, jnp.float32), jnp.full((_RT,), _BIGI, jnp.int32),
            jnp.full((_RT,), _INF, jnp.float32), jnp.full((_RT,), _BIGI, jnp.int32))
    ak1, aj1, ak2, aj2, ak3, aj3 = lax.fori_loop(0, nnz_s[c, t], body, init)

    count = cnt[0, 0]
    repeats = jnp.minimum(3, count)
    s1 = (ak1 < _INF) & (repeats >= 1)
    s2 = (ak2 < _INF) & (repeats >= 2)
    s3 = (ak3 < _INF) & (repeats >= 3)
    den = s1.astype(jnp.int32) + s2.astype(jnp.int32) + s3.astype(jnp.int32)
    keep = (rs > _SCORE_T) & (den > 0)
    j1o[0, 0, :] = jnp.where(s1, aj1, 0)
    j2o[0, 0, :] = jnp.where(s2, aj2, 0)
    j3o[0, 0, :] = jnp.where(s3, aj3, 0)
    s1o[0, 0, :] = s1.astype(jnp.int32)
    s2o[0, 0, :] = s2.astype(jnp.int32)
    s3o[0, 0, :] = s3.astype(jnp.int32)
    keepo[0, 0, :] = keep.astype(jnp.int32)


_CH = _MP // 4
_TAB = _M * 12


def _sc_posemean_body(pos_hbm, j1h, j2h, j3h, s1h, s2h, s3h, out_hbm,
                      tab_v, j1v, j2v, j3v, s1v, s2v, s3v, out_v):
    wid = lax.axis_index("s") * 2 + lax.axis_index("c")
    c = wid // 4
    q = wid % 4
    base = c * _MP + q * _CH
    pltpu.sync_copy(pos_hbm.at[pl.ds(c * _TAB, _TAB)], tab_v)
    pltpu.sync_copy(j1h.at[pl.ds(base, _CH)], j1v)
    pltpu.sync_copy(j2h.at[pl.ds(base, _CH)], j2v)
    pltpu.sync_copy(j3h.at[pl.ds(base, _CH)], j3v)
    pltpu.sync_copy(s1h.at[pl.ds(base, _CH)], s1v)
    pltpu.sync_copy(s2h.at[pl.ds(base, _CH)], s2v)
    pltpu.sync_copy(s3h.at[pl.ds(base, _CH)], s3v)

    lane = lax.iota(jnp.int32, 16)

    def body(g, carry):
        sl = pl.ds(g * 16, 16)
        jv1 = j1v[sl] * 12
        jv2 = j2v[sl] * 12
        jv3 = j3v[sl] * 12
        sv1 = s1v[sl]
        sv2 = s2v[sl]
        sv3 = s3v[sl]
        den = sv1 + sv2 + sv3
        rden = jnp.where(den == 0.0, 1.0, den)
        zero = den == 0.0
        lv = (g * 16 + lane) * 12
        for d in range(12):
            p1 = plsc.load_gather(tab_v, [jv1 + d])
            p2 = plsc.load_gather(tab_v, [jv2 + d])
            p3 = plsc.load_gather(tab_v, [jv3 + d])
            p = (p1 * sv1 + p2 * sv2 + p3 * sv3) / rden
            p = jnp.where(zero, 0.0, p)
            plsc.store_scatter(out_v, [lv + d], p)
        return carry

    lax.fori_loop(0, _CH // 16, body, 0)
    pltpu.sync_copy(out_v, out_hbm.at[pl.ds(base * 12, _CH * 12)])


@functools.partial(
    pl.kernel,
    out_type=jax.ShapeDtypeStruct((_C * _MP * 12,), jnp.float32),
    mesh=plsc.VectorSubcoreMesh(core_axis_name="c", subcore_axis_name="s"),
    compiler_params=pltpu.CompilerParams(needs_layout_passes=False),
    scratch_types=[
        pltpu.VMEM((_TAB,), jnp.float32),
        pltpu.VMEM((_CH,), jnp.int32),
        pltpu.VMEM((_CH,), jnp.int32),
        pltpu.VMEM((_CH,), jnp.int32),
        pltpu.VMEM((_CH,), jnp.float32),
        pltpu.VMEM((_CH,), jnp.float32),
        pltpu.VMEM((_CH,), jnp.float32),
        pltpu.VMEM((_CH * 12,), jnp.float32),
    ],
)
def _sc_posemean(pos_hbm, j1h, j2h, j3h, s1h, s2h, s3h, out_hbm,
                 tab_v, j1v, j2v, j3v, s1v, s2v, s3v, out_v):
    _sc_posemean_body(pos_hbm, j1h, j2h, j3h, s1h, s2h, s3h, out_hbm,
                      tab_v, j1v, j2v, j3v, s1v, s2v, s3v, out_v)


def _tile3(spec_kind):
    return pl.BlockSpec((1, 1, _RT), lambda c, t: (c * _NT + t, 0, 0))


def _col_spec(shape):
    return pl.BlockSpec(shape, lambda c, t: (c, 0, 0))


def _r3(x):
    return x.reshape(_C * _NT, 1, _RT)


def kernel(boxes3D, classification, poses, confidence):
    M, C = _M, _C
    cls = classification.reshape(M, C)
    b3 = boxes3D.reshape(M, C, 16)
    pos = poses.reshape(M, C, 12)
    cf = confidence.reshape(M, C)

    scores_t = jnp.full((C, _MP), -1.0, jnp.float32).at[:, :M].set(cls.T)
    ev = b3[:, :, 0::2]
    od = b3[:, :, 1::2]
    pad = jnp.zeros((C, _MP - M), jnp.float32)

    def padt(x):
        return jnp.concatenate([x.T, pad], axis=1)

    xmin = padt(jnp.min(ev, axis=2))
    ymin = padt(jnp.min(od, axis=2))
    xmax = padt(jnp.max(ev, axis=2))
    ymax = padt(jnp.max(od, axis=2))
    cg = padt(cf)
    counts = jnp.sum(cls > _SCORE_T, axis=0).astype(jnp.int32)

    col = lambda x: x[:, :, None]
    rows = _r3

    grid = (C, _NT)
    heads = pl.pallas_call(
        _heads_body,
        grid=grid,
        in_specs=[_tile3("r")] * 5 + [_col_spec((None, _MP, 1))] * 5,
        out_specs=_tile3("o"),
        out_shape=jax.ShapeDtypeStruct((C * _NT, 1, _RT), jnp.int32),
    )(rows(scores_t), rows(xmin), rows(ymin), rows(xmax), rows(ymax),
      col(scores_t), col(xmin), col(ymin), col(xmax), col(ymax))

    head_c = heads.reshape(C, _MP)

    t_of = head_c >> 8
    t_blocks = t_of.reshape(C, _NB, _CB)
    hist = jnp.sum(t_blocks[:, :, :, None] == jnp.arange(_NT)[None, None, None, :],
                   axis=2)
    mask_b = (hist > 0).transpose(0, 2, 1)
    nnz = jnp.sum(mask_b, axis=2).astype(jnp.int32)
    blist = jnp.sort(jnp.where(mask_b, jnp.arange(_NB)[None, None, :], _NB),
                     axis=2).astype(jnp.int32)

    outs3 = [jax.ShapeDtypeStruct((C * _NT, 1, _RT), jnp.int32)] * 7
    sel = pl.pallas_call(
        _select_body,
        grid=grid,
        in_specs=[_tile3("r"), _col_spec((None, _MP, 1)), _col_spec((None, _MP, 1)),
                  pl.BlockSpec((None, 1, 1), lambda c, t: (c, 0, 0)),
                  pl.BlockSpec(memory_space=pltpu.SMEM),
                  pl.BlockSpec(memory_space=pltpu.SMEM)],
        out_specs=[_tile3("o")] * 7,
        out_shape=outs3,
    )(rows(scores_t), col(head_c), col(cg), counts.reshape(C, 1, 1), nnz, blist)
    j1, j2, j3, s1, s2, s3, keep = [x.reshape(C, _MP) for x in sel]
    keep = keep[:, :M]

    pos_flat = pos.transpose(1, 0, 2).reshape(-1)
    sf = lambda x: x.astype(jnp.float32).reshape(-1)
    pout = _sc_posemean(pos_flat,
                        j1.reshape(-1), j2.reshape(-1), j3.reshape(-1),
                        sf(s1), sf(s2), sf(s3))
    pout = pout.reshape(C, _MP, 12)[:, :M]

    kb = keep.astype(jnp.bool_)
    keep_all = jnp.concatenate([kb, (counts == 0)[:, None]], axis=1)
    sc = jnp.where(kb, cls.T[:, :M], -jnp.inf)
    sent_sc = jnp.full((C, 1), cls[M - 1, C - 1], jnp.float32)
    scores_all = jnp.concatenate([jnp.where(keep_all[:, :M], sc, -jnp.inf), sent_sc], axis=1)
    lab = jnp.broadcast_to(jnp.arange(C, dtype=jnp.int32)[:, None], (C, M))
    labels_all = jnp.concatenate([lab, jnp.full((C, 1), -1, jnp.int32)], axis=1)
    idxv = jnp.broadcast_to(jnp.arange(M, dtype=jnp.int32)[None, :], (C, M))
    idx_all = jnp.concatenate([idxv, jnp.full((C, 1), -1, jnp.int32)], axis=1)
    poses_all = jnp.concatenate([pout, jnp.full((C, 1, 12), -1.0, jnp.float32)], axis=1)

    keep_f = keep_all.reshape(-1)
    scores_f = scores_all.reshape(-1)
    labels_f = labels_all.reshape(-1)
    idx_f = idx_all.reshape(-1)
    poses_f = poses_all.reshape(-1, 12)

    kint = keep_f.astype(jnp.int32)
    total = jnp.sum(kint)
    masked = jnp.where(keep_f, scores_f, -jnp.inf)
    top, _ = lax.top_k(masked, _MAXDET)
    scores_out = jnp.where(jnp.arange(_MAXDET) < total, top, -1.0)
    rank = jnp.cumsum(kint) - kint
    slot = jnp.where(keep_f, rank, _MAXDET)
    labels_out = jnp.full((_MAXDET,), -1, jnp.int32).at[slot].set(labels_f, mode='drop')
    idx_out = jnp.full((_MAXDET,), -1, jnp.int32).at[slot].set(idx_f, mode='drop')
    poses_out = jnp.full((_MAXDET, 12), -1.0, jnp.float32).at[slot].set(poses_f, mode='drop')
    return scores_out, labels_out, poses_out, idx_out
